# Initial kernel scaffold; baseline (speedup 1.0000x reference)
import jax
import jax.numpy as jnp
from jax.experimental import pallas as pl


def kernel(sent_emb, adj_mask, sent_counts, W1, a1_src, a1_dst, b1, W2, a2_src, a2_dst, b2):
    raise NotImplementedError("write your pallas kernel here")


# speedup vs baseline: 4.0272x; 4.0272x over previous
import jax
import jax.numpy as jnp
from jax.experimental import pallas as pl
from jax.experimental.pallas import tpu as pltpu

_B, _S, _D_IN, _HID, _HEADS = 16, 512, 256, 256, 4
_DH = _HID // _HEADS
_NEG = -1e9


def _body(x_ref, adj_ref, counts_ref, W1_ref, As1_ref, Ad1_ref, b1_ref,
          W2_ref, As2_ref, Ad2_ref, b2_ref, out_ref):
    b = pl.program_id(0)
    n = counts_ref[b]
    x = x_ref[0]
    adjf = adj_ref[0].astype(jnp.float32)
    adjt = adjf.T
    jj = jax.lax.broadcasted_iota(jnp.int32, (_S, _S), 0)
    ii = jax.lax.broadcasted_iota(jnp.int32, (_S, _S), 1)
    m = ((adjt > 0.5) | (ii == jj)) & (ii < n) & (jj < n)
    mf = m.astype(jnp.float32)

    def gat(xin, W_ref, As_ref, Ad_ref, b_ref):
        h = jnp.dot(xin, W_ref[...], preferred_element_type=jnp.float32)
        al_s = jnp.dot(h, As_ref[...], preferred_element_type=jnp.float32)
        al_d = jnp.dot(h, Ad_ref[...], preferred_element_type=jnp.float32)
        al_sr = al_s.T
        outs = []
        for hd in range(_HEADS):
            e = al_d[:, hd:hd + 1] + al_sr[hd:hd + 1, :]
            e = jnp.where(e >= 0, e, 0.2 * e)
            logits = jnp.where(m, e, _NEG)
            mx = jnp.max(logits, axis=1, keepdims=True)
            ex = jnp.exp(logits - mx) * mf
            den = jnp.sum(ex, axis=1, keepdims=True)
            attn = ex / jnp.where(den == 0.0, 1.0, den)
            outs.append(jnp.dot(attn, h[:, hd * _DH:(hd + 1) * _DH],
                                preferred_element_type=jnp.float32))
        return jnp.concatenate(outs, axis=1) + b_ref[...]

    x1 = gat(x, W1_ref, As1_ref, Ad1_ref, b1_ref)
    x2 = gat(x1, W2_ref, As2_ref, Ad2_ref, b2_ref)
    validj = jax.lax.broadcasted_iota(jnp.int32, (_S, 1), 0) < n
    out_ref[0] = x2 * validj.astype(jnp.float32)


def _head_mat(a):
    k = jnp.arange(_HID)
    sel = (k[:, None] // _DH) == jnp.arange(_HEADS)[None, :]
    return a.reshape(_HID)[:, None] * sel.astype(a.dtype)


def kernel(sent_emb, adj_mask, sent_counts, W1, a1_src, a1_dst, b1,
           W2, a2_src, a2_dst, b2):
    As1, Ad1 = _head_mat(a1_src), _head_mat(a1_dst)
    As2, Ad2 = _head_mat(a2_src), _head_mat(a2_dst)
    full = lambda shape: pl.BlockSpec(shape, lambda b: (0,) * len(shape))
    out = pl.pallas_call(
        _body,
        grid=(_B,),
        in_specs=[
            pl.BlockSpec((1, _S, _D_IN), lambda b: (b, 0, 0)),
            pl.BlockSpec((1, _S, _S), lambda b: (b, 0, 0)),
            pl.BlockSpec(memory_space=pltpu.SMEM),
            full((_D_IN, _HID)),
            full((_HID, _HEADS)),
            full((_HID, _HEADS)),
            full((1, _HID)),
            full((_HID, _HID)),
            full((_HID, _HEADS)),
            full((_HID, _HEADS)),
            full((1, _HID)),
        ],
        out_specs=pl.BlockSpec((1, _S, _HID), lambda b: (b, 0, 0)),
        out_shape=jax.ShapeDtypeStruct((_B, _S, _HID), jnp.float32),
    )(sent_emb, adj_mask, sent_counts, W1, As1, Ad1, b1.reshape(1, _HID),
      W2, As2, Ad2, b2.reshape(1, _HID))
    return out
